# Initial kernel scaffold; baseline (speedup 1.0000x reference)
import jax
import jax.numpy as jnp
from jax.experimental import pallas as pl


def kernel(x, edge_index, W1, b1, W2, b2):
    raise NotImplementedError("write your pallas kernel here")


# speedup vs baseline: 16.3454x; 16.3454x over previous
import functools

import jax
import jax.numpy as jnp
from jax import lax
from jax.experimental import pallas as pl
from jax.experimental.pallas import tpu as pltpu
from jax.experimental.pallas import tpu_sc as plsc

N_NODES = 10000
NPAD = 10240
ROWS_PER_TILE = NPAD // 16
E = 320000
K = 128
NCHUNKS = E // K
NW = 32
DEG_LANES = 16


def _mesh():
    return plsc.VectorSubcoreMesh(core_axis_name="c", subcore_axis_name="s")


def _deg_body(dst_hbm, zeros_hbm, out_hbm, didx, obuf, acc, sem):
    c = lax.axis_index("c")
    s = lax.axis_index("s")
    wid = s * 2 + c

    def fill(i, carry):
        obuf[i, :] = jnp.full((16,), 1.0, jnp.float32)
        return carry

    lax.fori_loop(0, K, fill, 0)
    row0 = s * ROWS_PER_TILE
    pltpu.sync_copy(zeros_hbm.at[pl.ds(row0, ROWS_PER_TILE)],
                    acc.at[pl.ds(row0, ROWS_PER_TILE)])
    plsc.subcore_barrier()

    nch = 78 + jnp.where(wid < NCHUNKS - 78 * NW, 1, 0)

    def body(j, carry):
        ci = wid + j * NW
        pltpu.sync_copy(dst_hbm.at[pl.ds(ci * K, K)], didx)
        pltpu.sync_copy(obuf, acc.at[didx], add=True)
        return carry

    lax.fori_loop(0, nch, body, 0)
    plsc.subcore_barrier()
    pltpu.sync_copy(acc.at[pl.ds(row0, ROWS_PER_TILE)],
                    out_hbm.at[c].at[pl.ds(row0, ROWS_PER_TILE)])


def _degree_partials(dst, zeros_rows):
    kern = pl.kernel(
        _deg_body,
        out_type=jax.ShapeDtypeStruct((2, NPAD, DEG_LANES), jnp.float32),
        mesh=_mesh(),
        compiler_params=pltpu.CompilerParams(use_tc_tiling_on_sc=False),
        scratch_types=[
            pltpu.VMEM((K,), jnp.int32),
            pltpu.VMEM((K, DEG_LANES), jnp.float32),
            pltpu.VMEM_SHARED((NPAD, DEG_LANES), jnp.float32),
            pltpu.SemaphoreType.DMA,
        ],
    )
    return kern(dst, zeros_rows)


def _make_agg(D):
    def body(y_hbm, src_hbm, dst_hbm, out_hbm, sidx, didx, rows, acc, gsem):
        c = lax.axis_index("c")
        s = lax.axis_index("s")
        wid = s * 2 + c
        row0 = s * ROWS_PER_TILE
        pltpu.sync_copy(y_hbm.at[pl.ds(row0, ROWS_PER_TILE)],
                        acc.at[pl.ds(row0, ROWS_PER_TILE)])
        plsc.subcore_barrier()

        nch = 78 + jnp.where(wid < NCHUNKS - 78 * NW, 1, 0)

        def chunk(j, carry):
            ci = wid + j * NW
            pltpu.sync_copy(src_hbm.at[pl.ds(ci * K, K)], sidx)
            pltpu.async_copy(y_hbm.at[sidx], rows, gsem).wait()
            pltpu.sync_copy(dst_hbm.at[pl.ds(ci * K, K)], didx)
            pltpu.sync_copy(rows, acc.at[didx], add=True)
            return carry

        lax.fori_loop(0, nch, chunk, 0)
        plsc.subcore_barrier()
        pltpu.sync_copy(acc.at[pl.ds(row0, ROWS_PER_TILE)],
                        out_hbm.at[c].at[pl.ds(row0, ROWS_PER_TILE)])

    kern = pl.kernel(
        body,
        out_type=jax.ShapeDtypeStruct((2, NPAD, D), jnp.float32),
        mesh=_mesh(),
        compiler_params=pltpu.CompilerParams(use_tc_tiling_on_sc=False),
        scratch_types=[
            pltpu.VMEM((K,), jnp.int32),
            pltpu.VMEM((K,), jnp.int32),
            pltpu.VMEM((K, D), jnp.float32),
            pltpu.VMEM_SHARED((NPAD, D), jnp.float32),
            pltpu.SemaphoreType.DMA,
        ],
    )
    return kern


_RB = 640


def _prep1_body(x_ref, w_ref, degp_ref, y_ref, dinv_ref):
    deg = degp_ref[0, :, 0:1] + degp_ref[1, :, 0:1] + 1.0
    dinv = lax.rsqrt(deg)
    xw = jnp.dot(x_ref[...], w_ref[...], preferred_element_type=jnp.float32)
    y_ref[...] = xw * dinv
    dinv_ref[...] = dinv


def _prep1(x_pad, w1, degp):
    grid = NPAD // _RB
    return pl.pallas_call(
        _prep1_body,
        grid=(grid,),
        in_specs=[
            pl.BlockSpec((_RB, 128), lambda i: (i, 0)),
            pl.BlockSpec((128, 128), lambda i: (0, 0)),
            pl.BlockSpec((2, _RB, DEG_LANES), lambda i: (0, i, 0)),
        ],
        out_specs=[
            pl.BlockSpec((_RB, 128), lambda i: (i, 0)),
            pl.BlockSpec((_RB, 1), lambda i: (i, 0)),
        ],
        out_shape=[
            jax.ShapeDtypeStruct((NPAD, 128), jnp.float32),
            jax.ShapeDtypeStruct((NPAD, 1), jnp.float32),
        ],
    )(x_pad, w1, degp)


def _mid_body(p0_ref, p1_ref, y_ref, dinv_ref, b_ref, w_ref, y2_ref):
    dinv = dinv_ref[...]
    h = dinv * (p0_ref[...] + p1_ref[...] - y_ref[...]) + b_ref[...]
    h = jnp.maximum(h, 0.0)
    y2_ref[...] = jnp.dot(h, w_ref[...], preferred_element_type=jnp.float32) * dinv


def _mid(p0, p1, y1, dinv, b1, w2):
    grid = NPAD // _RB
    return pl.pallas_call(
        _mid_body,
        grid=(grid,),
        in_specs=[
            pl.BlockSpec((_RB, 128), lambda i: (i, 0)),
            pl.BlockSpec((_RB, 128), lambda i: (i, 0)),
            pl.BlockSpec((_RB, 128), lambda i: (i, 0)),
            pl.BlockSpec((_RB, 1), lambda i: (i, 0)),
            pl.BlockSpec((1, 128), lambda i: (0, 0)),
            pl.BlockSpec((128, 64), lambda i: (0, 0)),
        ],
        out_specs=pl.BlockSpec((_RB, 64), lambda i: (i, 0)),
        out_shape=jax.ShapeDtypeStruct((NPAD, 64), jnp.float32),
    )(p0, p1, y1, dinv, b1, w2)


def _final_body(p0_ref, p1_ref, y_ref, dinv_ref, b_ref, o_ref):
    t = dinv_ref[...] * (p0_ref[...] + p1_ref[...] - y_ref[...]) + b_ref[...]
    m = jnp.max(t, axis=1, keepdims=True)
    e = t - m
    lse = jnp.log(jnp.sum(jnp.exp(e), axis=1, keepdims=True))
    o_ref[...] = e - lse


def _final(p0, p1, y2, dinv, b2):
    grid = NPAD // _RB
    return pl.pallas_call(
        _final_body,
        grid=(grid,),
        in_specs=[
            pl.BlockSpec((_RB, 64), lambda i: (i, 0)),
            pl.BlockSpec((_RB, 64), lambda i: (i, 0)),
            pl.BlockSpec((_RB, 64), lambda i: (i, 0)),
            pl.BlockSpec((_RB, 1), lambda i: (i, 0)),
            pl.BlockSpec((1, 64), lambda i: (0, 0)),
        ],
        out_specs=pl.BlockSpec((_RB, 64), lambda i: (i, 0)),
        out_shape=jax.ShapeDtypeStruct((NPAD, 64), jnp.float32),
    )(p0, p1, y2, dinv, b2)


@jax.jit
def kernel(x, edge_index, W1, b1, W2, b2):
    ei = edge_index.astype(jnp.int32)
    src = ei[0]
    dst = ei[1]
    x_pad = jnp.pad(x, ((0, NPAD - N_NODES), (0, 0)))
    zeros_rows = jnp.zeros((NPAD, DEG_LANES), jnp.float32)

    degp = _degree_partials(dst, zeros_rows)
    y1, dinv = _prep1(x_pad, W1, degp)

    agg128 = _make_agg(128)
    p1 = agg128(y1, src, dst)
    y2 = _mid(p1[0], p1[1], y1, dinv, b1.reshape(1, 128), W2)

    agg64 = _make_agg(64)
    p2 = agg64(y2, src, dst)
    out = _final(p2[0], p2[1], y2, dinv, b2.reshape(1, 64))
    return out[:N_NODES]


# speedup vs baseline: 28.3695x; 1.7356x over previous
import functools

import jax
import jax.numpy as jnp
from jax import lax
from jax.experimental import pallas as pl
from jax.experimental.pallas import tpu as pltpu
from jax.experimental.pallas import tpu_sc as plsc

N_NODES = 10000
NPAD = 10240
ROWS_PER_TILE = NPAD // 16
E = 320000
K = 128
NCHUNKS = E // K
NW = 32
DEG_LANES = 16


def _mesh():
    return plsc.VectorSubcoreMesh(core_axis_name="c", subcore_axis_name="s")


def _deg_body(dst_hbm, zeros_hbm, out_hbm, didx, obuf, acc, sem):
    c = lax.axis_index("c")
    s = lax.axis_index("s")
    wid = s * 2 + c

    def fill(i, carry):
        obuf[i, :] = jnp.full((16,), 1.0, jnp.float32)
        return carry

    lax.fori_loop(0, K, fill, 0)
    row0 = s * ROWS_PER_TILE
    pltpu.sync_copy(zeros_hbm.at[pl.ds(row0, ROWS_PER_TILE)],
                    acc.at[pl.ds(row0, ROWS_PER_TILE)])
    plsc.subcore_barrier()

    nch = 78 + jnp.where(wid < NCHUNKS - 78 * NW, 1, 0)

    def body(j, carry):
        ci = wid + j * NW
        pltpu.sync_copy(dst_hbm.at[pl.ds(ci * K, K)], didx)
        pltpu.sync_copy(obuf, acc.at[didx], add=True)
        return carry

    lax.fori_loop(0, nch, body, 0)
    plsc.subcore_barrier()
    pltpu.sync_copy(acc.at[pl.ds(row0, ROWS_PER_TILE)],
                    out_hbm.at[c].at[pl.ds(row0, ROWS_PER_TILE)])


def _degree_partials(dst, zeros_rows):
    kern = pl.kernel(
        _deg_body,
        out_type=jax.ShapeDtypeStruct((2, NPAD, DEG_LANES), jnp.float32),
        mesh=_mesh(),
        compiler_params=pltpu.CompilerParams(use_tc_tiling_on_sc=False),
        scratch_types=[
            pltpu.VMEM((K,), jnp.int32),
            pltpu.VMEM((K, DEG_LANES), jnp.float32),
            pltpu.VMEM_SHARED((NPAD, DEG_LANES), jnp.float32),
            pltpu.SemaphoreType.DMA,
        ],
    )
    return kern(dst, zeros_rows)


_CPT = NCHUNKS // NW
_REM = NCHUNKS - _CPT * NW


def _make_agg(D):
    def body(y_hbm, src2_hbm, dst2_hbm, out_hbm,
             sidx_all, didx0, didx1, rows0, rows1,
             acc, isem, gsem0, gsem1, dsem0, dsem1, tsem):
        c = lax.axis_index("c")
        s = lax.axis_index("s")
        wid = s * 2 + c
        row0 = s * ROWS_PER_TILE
        c0 = wid * _CPT
        i1 = pltpu.async_copy(src2_hbm.at[pl.ds(c0, _CPT)], sidx_all, isem)
        pltpu.sync_copy(y_hbm.at[pl.ds(row0, ROWS_PER_TILE)],
                        acc.at[pl.ds(row0, ROWS_PER_TILE)])
        plsc.subcore_barrier()
        i1.wait()

        def wait_d0():
            pltpu.make_async_copy(dst2_hbm.at[c0], didx0, dsem0).wait()

        def wait_d1():
            pltpu.make_async_copy(dst2_hbm.at[c0], didx1, dsem1).wait()

        pltpu.async_copy(dst2_hbm.at[c0], didx0, dsem0)
        pltpu.async_copy(dst2_hbm.at[c0 + 1], didx1, dsem1)
        pltpu.async_copy(y_hbm.at[sidx_all.at[0]], rows0, gsem0)

        def pair(p, carry):
            ja = 2 * p
            pltpu.async_copy(y_hbm.at[sidx_all.at[ja + 1]], rows1, gsem1)
            pltpu.make_async_copy(y_hbm.at[sidx_all.at[ja]], rows0, gsem0).wait()
            wait_d0()
            pltpu.sync_copy(rows0, acc.at[didx0], add=True)

            @pl.when(p < _CPT // 2 - 1)
            def _():
                pltpu.async_copy(y_hbm.at[sidx_all.at[ja + 2]], rows0, gsem0)
                pltpu.async_copy(dst2_hbm.at[c0 + ja + 2], didx0, dsem0)

            pltpu.make_async_copy(y_hbm.at[sidx_all.at[ja + 1]], rows1,
                                  gsem1).wait()
            wait_d1()
            pltpu.sync_copy(rows1, acc.at[didx1], add=True)

            @pl.when(p < _CPT // 2 - 1)
            def _():
                pltpu.async_copy(dst2_hbm.at[c0 + ja + 3], didx1, dsem1)

            return carry

        lax.fori_loop(0, _CPT // 2, pair, 0)

        @pl.when(wid < _REM)
        def _():
            ct = _CPT * NW + wid
            pltpu.sync_copy(src2_hbm.at[ct], didx0)
            pltpu.sync_copy(dst2_hbm.at[ct], didx1)
            pltpu.async_copy(y_hbm.at[didx0], rows0, tsem).wait()
            pltpu.sync_copy(rows0, acc.at[didx1], add=True)

        plsc.subcore_barrier()
        pltpu.sync_copy(acc.at[pl.ds(row0, ROWS_PER_TILE)],
                        out_hbm.at[c].at[pl.ds(row0, ROWS_PER_TILE)])

    kern = pl.kernel(
        body,
        out_type=jax.ShapeDtypeStruct((2, NPAD, D), jnp.float32),
        mesh=_mesh(),
        compiler_params=pltpu.CompilerParams(use_tc_tiling_on_sc=False),
        scratch_types=[
            pltpu.VMEM((_CPT, K), jnp.int32),
            pltpu.VMEM((K,), jnp.int32),
            pltpu.VMEM((K,), jnp.int32),
            pltpu.VMEM((K, D), jnp.float32),
            pltpu.VMEM((K, D), jnp.float32),
            pltpu.VMEM_SHARED((NPAD, D), jnp.float32),
            pltpu.SemaphoreType.DMA,
            pltpu.SemaphoreType.DMA,
            pltpu.SemaphoreType.DMA,
            pltpu.SemaphoreType.DMA,
            pltpu.SemaphoreType.DMA,
            pltpu.SemaphoreType.DMA,
        ],
    )
    return kern


_RB = 640


def _prep1_body(x_ref, w_ref, degp_ref, y_ref, dinv_ref):
    deg = degp_ref[0, :, 0:1] + degp_ref[1, :, 0:1] + 1.0
    dinv = lax.rsqrt(deg)
    xw = jnp.dot(x_ref[...], w_ref[...], preferred_element_type=jnp.float32)
    y_ref[...] = xw * dinv
    dinv_ref[...] = dinv


def _prep1(x_pad, w1, degp):
    grid = NPAD // _RB
    return pl.pallas_call(
        _prep1_body,
        grid=(grid,),
        in_specs=[
            pl.BlockSpec((_RB, 128), lambda i: (i, 0)),
            pl.BlockSpec((128, 128), lambda i: (0, 0)),
            pl.BlockSpec((2, _RB, DEG_LANES), lambda i: (0, i, 0)),
        ],
        out_specs=[
            pl.BlockSpec((_RB, 128), lambda i: (i, 0)),
            pl.BlockSpec((_RB, 1), lambda i: (i, 0)),
        ],
        out_shape=[
            jax.ShapeDtypeStruct((NPAD, 128), jnp.float32),
            jax.ShapeDtypeStruct((NPAD, 1), jnp.float32),
        ],
    )(x_pad, w1, degp)


def _mid_body(p0_ref, p1_ref, y_ref, dinv_ref, b_ref, w_ref, y2_ref):
    dinv = dinv_ref[...]
    h = dinv * (p0_ref[...] + p1_ref[...] - y_ref[...]) + b_ref[...]
    h = jnp.maximum(h, 0.0)
    y2_ref[...] = jnp.dot(h, w_ref[...], preferred_element_type=jnp.float32) * dinv


def _mid(p0, p1, y1, dinv, b1, w2):
    grid = NPAD // _RB
    return pl.pallas_call(
        _mid_body,
        grid=(grid,),
        in_specs=[
            pl.BlockSpec((_RB, 128), lambda i: (i, 0)),
            pl.BlockSpec((_RB, 128), lambda i: (i, 0)),
            pl.BlockSpec((_RB, 128), lambda i: (i, 0)),
            pl.BlockSpec((_RB, 1), lambda i: (i, 0)),
            pl.BlockSpec((1, 128), lambda i: (0, 0)),
            pl.BlockSpec((128, 64), lambda i: (0, 0)),
        ],
        out_specs=pl.BlockSpec((_RB, 64), lambda i: (i, 0)),
        out_shape=jax.ShapeDtypeStruct((NPAD, 64), jnp.float32),
    )(p0, p1, y1, dinv, b1, w2)


def _final_body(p0_ref, p1_ref, y_ref, dinv_ref, b_ref, o_ref):
    t = dinv_ref[...] * (p0_ref[...] + p1_ref[...] - y_ref[...]) + b_ref[...]
    m = jnp.max(t, axis=1, keepdims=True)
    e = t - m
    lse = jnp.log(jnp.sum(jnp.exp(e), axis=1, keepdims=True))
    o_ref[...] = e - lse


def _final(p0, p1, y2, dinv, b2):
    grid = NPAD // _RB
    return pl.pallas_call(
        _final_body,
        grid=(grid,),
        in_specs=[
            pl.BlockSpec((_RB, 64), lambda i: (i, 0)),
            pl.BlockSpec((_RB, 64), lambda i: (i, 0)),
            pl.BlockSpec((_RB, 64), lambda i: (i, 0)),
            pl.BlockSpec((_RB, 1), lambda i: (i, 0)),
            pl.BlockSpec((1, 64), lambda i: (0, 0)),
        ],
        out_specs=pl.BlockSpec((_RB, 64), lambda i: (i, 0)),
        out_shape=jax.ShapeDtypeStruct((NPAD, 64), jnp.float32),
    )(p0, p1, y2, dinv, b2)


@jax.jit
def kernel(x, edge_index, W1, b1, W2, b2):
    ei = edge_index.astype(jnp.int32)
    src = ei[0]
    dst = ei[1]
    src2 = src.reshape(NCHUNKS, K)
    dst2 = dst.reshape(NCHUNKS, K)
    x_pad = jnp.pad(x, ((0, NPAD - N_NODES), (0, 0)))
    zeros_rows = jnp.zeros((NPAD, DEG_LANES), jnp.float32)

    degp = _degree_partials(dst, zeros_rows)
    y1, dinv = _prep1(x_pad, W1, degp)

    agg128 = _make_agg(128)
    p1 = agg128(y1, src2, dst2)
    y2 = _mid(p1[0], p1[1], y1, dinv, b1.reshape(1, 128), W2)

    agg64 = _make_agg(64)
    p2 = agg64(y2, src2, dst2)
    out = _final(p2[0], p2[1], y2, dinv, b2.reshape(1, 64))
    return out[:N_NODES]
